# baseline (speedup 1.0000x reference)
import functools

import jax
import jax.numpy as jnp
from jax import lax
from jax.experimental import pallas as pl
from jax.experimental.pallas import tpu as pltpu
from jax.experimental.pallas import tpu_sc as plsc

N_BINS = 256
HIDDEN_DIM = 64
L = 16

_info = plsc.get_sparse_core_info()
NUM_CORES = _info.num_cores
NUM_SUBCORES = _info.num_subcores
NW = NUM_CORES * NUM_SUBCORES

CHUNK = 640
GATHER = 128
N_GATHERS = CHUNK // GATHER
NBUF = 2

IBLK = 128


def _sc_gather(total, b_per_w, n_chunks):
    mesh = plsc.VectorSubcoreMesh(core_axis_name="c", subcore_axis_name="s")
    n_outer = n_chunks // NBUF

    @functools.partial(
        pl.kernel,
        mesh=mesh,
        compiler_params=pltpu.CompilerParams(use_tc_tiling_on_sc=False),
        out_type=jax.ShapeDtypeStruct((total, HIDDEN_DIM), jnp.bfloat16),
        scratch_types=[
            pltpu.VMEM((NBUF, CHUNK), jnp.float32),
            pltpu.VMEM((NBUF, CHUNK), jnp.int32),
            pltpu.VMEM((NBUF, CHUNK, HIDDEN_DIM), jnp.bfloat16),
            pltpu.VMEM((L,), jnp.float32),
            pltpu.VMEM((L,), jnp.float32),
            pltpu.VMEM_SHARED((N_BINS, HIDDEN_DIM), jnp.bfloat16),
            pltpu.SemaphoreType.DMA,
            pltpu.SemaphoreType.DMA,
            pltpu.SemaphoreType.DMA,
            pltpu.SemaphoreType.DMA,
        ],
    )
    def k(energy_hbm, table_hbm, emin_hbm, scale_hbm, out_hbm,
          ev, idxv, rows, eminv, scalev, tloc,
          semg0, semg1, semo0, semo1):
        semg = (semg0, semg1)
        semo = (semo0, semo1)
        sid = lax.axis_index("s")
        wid = sid * NUM_CORES + lax.axis_index("c")

        @pl.when(sid == 0)
        def _stage_table():
            pltpu.sync_copy(table_hbm, tloc)

        plsc.subcore_barrier()
        pltpu.sync_copy(emin_hbm, eminv)
        pltpu.sync_copy(scale_hbm, scalev)
        emin = eminv[...]
        scale = scalev[...]
        wbase = wid * b_per_w

        def fire_gathers(b, c):
            base = wbase + c * CHUNK
            pltpu.sync_copy(energy_hbm.at[pl.ds(base, CHUNK)], ev.at[b])
            for l in range(CHUNK // L):
                e = ev[b, pl.ds(l * L, L)]
                t = (e - emin) * scale
                t = jnp.minimum(jnp.maximum(t, 0.0), float(N_BINS - 1))
                idxv[b, pl.ds(l * L, L)] = t.astype(jnp.int32)
            for j in range(N_GATHERS):
                pltpu.async_copy(
                    tloc.at[idxv.at[b].at[pl.ds(j * GATHER, GATHER)]],
                    rows.at[b].at[pl.ds(j * GATHER, GATHER)],
                    semg[b])

        def wait_gathers(b):
            for j in range(N_GATHERS):
                pltpu.make_async_copy(
                    tloc.at[idxv.at[b].at[pl.ds(j * GATHER, GATHER)]],
                    rows.at[b].at[pl.ds(j * GATHER, GATHER)],
                    semg[b]).wait()

        def fire_out(b, c):
            base = wbase + c * CHUNK
            pltpu.async_copy(rows.at[b], out_hbm.at[pl.ds(base, CHUNK)],
                             semo[b])

        def wait_out(b):
            pltpu.make_async_copy(rows.at[b], out_hbm.at[pl.ds(wbase, CHUNK)],
                                  semo[b]).wait()

        @pl.loop(0, n_outer)
        def _outer(o):
            for b in range(NBUF):
                c = o * NBUF + b
                pb = 1 - b

                @pl.when(o > 0)
                def _reclaim():
                    wait_out(b)

                fire_gathers(b, c)

                if b == 0:
                    @pl.when(o > 0)
                    def _drain_prev():
                        wait_gathers(pb)
                        fire_out(pb, c - 1)
                else:
                    wait_gathers(pb)
                    fire_out(pb, c - 1)

        wait_gathers(1)
        fire_out(1, n_chunks - 1)
        wait_out(0)
        wait_out(1)

    return k


def _tc_transpose(nrows, ncol):
    ncol2 = ncol // 2
    rows_per_blk = IBLK * ncol * HIDDEN_DIM // 128

    def body(x_ref, o_ref):
        x = x_ref[...]
        w = x.reshape(IBLK, ncol2 * 128)
        o_ref[...] = w.T.reshape(ncol, HIDDEN_DIM, IBLK).astype(jnp.float32)

    return pl.pallas_call(
        body,
        grid=(nrows // IBLK,),
        in_specs=[pl.BlockSpec((rows_per_blk, 128), lambda s: (s, 0))],
        out_specs=pl.BlockSpec((ncol, HIDDEN_DIM, IBLK), lambda s: (0, 0, s)),
        out_shape=jax.ShapeDtypeStruct((ncol, HIDDEN_DIM, nrows), jnp.float32),
    )


def kernel(energy, table, energy_min, energy_max):
    nrows, ncol = energy.shape
    total = nrows * ncol
    b_per_w = total // NW
    n_chunks = b_per_w // CHUNK
    assert b_per_w * NW == total and n_chunks * CHUNK == b_per_w
    assert n_chunks % NBUF == 0

    scale = jnp.float32(N_BINS - 1) / (energy_max - energy_min + jnp.float32(1e-8))
    emin16 = jnp.full((L,), energy_min, dtype=jnp.float32)
    scale16 = jnp.full((L,), scale, dtype=jnp.float32)

    flat = _sc_gather(total, b_per_w, n_chunks)(
        energy.reshape(-1), table.astype(jnp.bfloat16), emin16, scale16)
    twod = flat.reshape(total * HIDDEN_DIM // 128, 128)
    b = _tc_transpose(nrows, ncol)(twod)
    return b.transpose(2, 0, 1)


# speedup vs baseline: 2.4580x; 2.4580x over previous
import functools

import jax
import jax.numpy as jnp
from jax import lax
from jax.experimental import pallas as pl
from jax.experimental.pallas import tpu as pltpu
from jax.experimental.pallas import tpu_sc as plsc

N_BINS = 256
HIDDEN_DIM = 64
L = 16

_info = plsc.get_sparse_core_info()
NUM_CORES = _info.num_cores
NUM_SUBCORES = _info.num_subcores
NW = NUM_CORES * NUM_SUBCORES

CHUNK = 800
GATHER = 128
_SPLITS = []
_off = 0
while _off < CHUNK:
    _SPLITS.append((_off, min(GATHER, CHUNK - _off)))
    _off += GATHER
NBUF = 2

IBLK = 256


def _sc_gather(total, b_per_w, n_chunks):
    mesh = plsc.VectorSubcoreMesh(core_axis_name="c", subcore_axis_name="s")
    n_outer = n_chunks // NBUF

    @functools.partial(
        pl.kernel,
        mesh=mesh,
        compiler_params=pltpu.CompilerParams(use_tc_tiling_on_sc=False),
        out_type=jax.ShapeDtypeStruct((total, HIDDEN_DIM), jnp.float32),
        scratch_types=[
            pltpu.VMEM((NBUF, CHUNK), jnp.float32),
            pltpu.VMEM((NBUF, CHUNK), jnp.int32),
            pltpu.VMEM((NBUF, CHUNK, HIDDEN_DIM), jnp.float32),
            pltpu.VMEM((L,), jnp.float32),
            pltpu.VMEM((L,), jnp.float32),
            pltpu.VMEM_SHARED((N_BINS, HIDDEN_DIM), jnp.float32),
            pltpu.SemaphoreType.DMA,
            pltpu.SemaphoreType.DMA,
            pltpu.SemaphoreType.DMA,
            pltpu.SemaphoreType.DMA,
        ],
    )
    def k(energy_hbm, table_hbm, emin_hbm, scale_hbm, out_hbm,
          ev, idxv, rows, eminv, scalev, tloc,
          semg0, semg1, semo0, semo1):
        semg = (semg0, semg1)
        semo = (semo0, semo1)
        sid = lax.axis_index("s")
        wid = sid * NUM_CORES + lax.axis_index("c")

        @pl.when(sid == 0)
        def _stage_table():
            pltpu.sync_copy(table_hbm, tloc)

        plsc.subcore_barrier()
        pltpu.sync_copy(emin_hbm, eminv)
        pltpu.sync_copy(scale_hbm, scalev)
        emin = eminv[...]
        scale = scalev[...]
        wbase = wid * b_per_w

        def fire_gathers(b, c):
            base = wbase + c * CHUNK
            pltpu.sync_copy(energy_hbm.at[pl.ds(base, CHUNK)], ev.at[b])
            for l in range(CHUNK // L):
                e = ev[b, pl.ds(l * L, L)]
                t = (e - emin) * scale
                t = jnp.minimum(jnp.maximum(t, 0.0), float(N_BINS - 1))
                idxv[b, pl.ds(l * L, L)] = t.astype(jnp.int32)
            for (off, n) in _SPLITS:
                pltpu.async_copy(
                    tloc.at[idxv.at[b].at[pl.ds(off, n)]],
                    rows.at[b].at[pl.ds(off, n)],
                    semg[b])

        def wait_gathers(b):
            for (off, n) in _SPLITS:
                pltpu.make_async_copy(
                    tloc.at[idxv.at[b].at[pl.ds(off, n)]],
                    rows.at[b].at[pl.ds(off, n)],
                    semg[b]).wait()

        def fire_out(b, c):
            base = wbase + c * CHUNK
            pltpu.async_copy(rows.at[b], out_hbm.at[pl.ds(base, CHUNK)],
                             semo[b])

        def wait_out(b):
            pltpu.make_async_copy(rows.at[b], out_hbm.at[pl.ds(wbase, CHUNK)],
                                  semo[b]).wait()

        @pl.loop(0, n_outer)
        def _outer(o):
            for b in range(NBUF):
                c = o * NBUF + b
                pb = 1 - b

                @pl.when(o > 0)
                def _reclaim():
                    wait_out(b)

                fire_gathers(b, c)

                if b == 0:
                    @pl.when(o > 0)
                    def _drain_prev():
                        wait_gathers(pb)
                        fire_out(pb, c - 1)
                else:
                    wait_gathers(pb)
                    fire_out(pb, c - 1)

        wait_gathers(1)
        fire_out(1, n_chunks - 1)
        wait_out(0)
        wait_out(1)

    return k


def _tc_transpose(nrows, ncol):
    ncol2 = ncol // 2
    rows_per_blk = IBLK * ncol * HIDDEN_DIM // 128

    def body(x_ref, o_ref):
        x = x_ref[...]
        w = x.reshape(IBLK, ncol2 * 128)
        o_ref[...] = w.T.reshape(ncol, HIDDEN_DIM, IBLK)

    return pl.pallas_call(
        body,
        grid=(nrows // IBLK,),
        in_specs=[pl.BlockSpec((rows_per_blk, 128), lambda s: (s, 0))],
        out_specs=pl.BlockSpec((ncol, HIDDEN_DIM, IBLK), lambda s: (0, 0, s)),
        out_shape=jax.ShapeDtypeStruct((ncol, HIDDEN_DIM, nrows), jnp.float32),
    )


def kernel(energy, table, energy_min, energy_max):
    nrows, ncol = energy.shape
    total = nrows * ncol
    b_per_w = total // NW
    n_chunks = b_per_w // CHUNK
    assert b_per_w * NW == total and n_chunks * CHUNK == b_per_w
    assert n_chunks % NBUF == 0

    scale = jnp.float32(N_BINS - 1) / (energy_max - energy_min + jnp.float32(1e-8))
    emin16 = jnp.full((L,), energy_min, dtype=jnp.float32)
    scale16 = jnp.full((L,), scale, dtype=jnp.float32)

    flat = _sc_gather(total, b_per_w, n_chunks)(
        energy.reshape(-1), table, emin16, scale16)
    twod = flat.reshape(total * HIDDEN_DIM // 128, 128)
    b = _tc_transpose(nrows, ncol)(twod)
    return b.transpose(2, 0, 1)


# speedup vs baseline: 2.5614x; 1.0421x over previous
import functools

import jax
import jax.numpy as jnp
from jax import lax
from jax.experimental import pallas as pl
from jax.experimental.pallas import tpu as pltpu
from jax.experimental.pallas import tpu_sc as plsc

N_BINS = 256
HIDDEN_DIM = 64
L = 16

_info = plsc.get_sparse_core_info()
NUM_CORES = _info.num_cores
NUM_SUBCORES = _info.num_subcores
NW = NUM_CORES * NUM_SUBCORES

CHUNK = 800
GATHER = 128
_SPLITS = []
_off = 0
while _off < CHUNK:
    _SPLITS.append((_off, min(GATHER, CHUNK - _off)))
    _off += GATHER
NBUF = 2

IBLK = 256


def _sc_gather(total, b_per_w, n_chunks):
    mesh = plsc.VectorSubcoreMesh(core_axis_name="c", subcore_axis_name="s")
    n_outer = n_chunks // NBUF

    @functools.partial(
        pl.kernel,
        mesh=mesh,
        compiler_params=pltpu.CompilerParams(use_tc_tiling_on_sc=False),
        out_type=jax.ShapeDtypeStruct((total, HIDDEN_DIM), jnp.float32),
        scratch_types=[
            pltpu.VMEM((NBUF, CHUNK), jnp.float32),
            pltpu.VMEM((NBUF, CHUNK), jnp.int32),
            pltpu.VMEM((NBUF, CHUNK, HIDDEN_DIM), jnp.float32),
            pltpu.VMEM((L,), jnp.float32),
            pltpu.VMEM((L,), jnp.float32),
            pltpu.VMEM_SHARED((N_BINS, HIDDEN_DIM), jnp.float32),
            pltpu.SemaphoreType.DMA,
            pltpu.SemaphoreType.DMA,
            pltpu.SemaphoreType.DMA,
            pltpu.SemaphoreType.DMA,
            pltpu.SemaphoreType.DMA,
            pltpu.SemaphoreType.DMA,
        ],
    )
    def k(energy_hbm, table_hbm, emin_hbm, scale_hbm, out_hbm,
          ev, idxv, rows, eminv, scalev, tloc,
          semg0, semg1, semo0, semo1, seme0, seme1):
        semg = (semg0, semg1)
        semo = (semo0, semo1)
        seme = (seme0, seme1)
        sid = lax.axis_index("s")
        wid = sid * NUM_CORES + lax.axis_index("c")

        @pl.when(sid == 0)
        def _stage_table():
            pltpu.sync_copy(table_hbm, tloc)

        plsc.subcore_barrier()
        pltpu.sync_copy(emin_hbm, eminv)
        pltpu.sync_copy(scale_hbm, scalev)
        emin = eminv[...]
        scale = scalev[...]
        wbase = wid * b_per_w

        def fire_energy(b, c):
            base = wbase + c * CHUNK
            pltpu.async_copy(energy_hbm.at[pl.ds(base, CHUNK)], ev.at[b],
                             seme[b])

        def wait_energy(b):
            pltpu.make_async_copy(energy_hbm.at[pl.ds(wbase, CHUNK)],
                                  ev.at[b], seme[b]).wait()

        def fire_gathers(b, c):
            for l in range(CHUNK // L):
                e = ev[b, pl.ds(l * L, L)]
                t = (e - emin) * scale
                t = jnp.minimum(jnp.maximum(t, 0.0), float(N_BINS - 1))
                idxv[b, pl.ds(l * L, L)] = t.astype(jnp.int32)
            for (off, n) in _SPLITS:
                pltpu.async_copy(
                    tloc.at[idxv.at[b].at[pl.ds(off, n)]],
                    rows.at[b].at[pl.ds(off, n)],
                    semg[b])

        def wait_gathers(b):
            for (off, n) in _SPLITS:
                pltpu.make_async_copy(
                    tloc.at[idxv.at[b].at[pl.ds(off, n)]],
                    rows.at[b].at[pl.ds(off, n)],
                    semg[b]).wait()

        def fire_out(b, c):
            base = wbase + c * CHUNK
            pltpu.async_copy(rows.at[b], out_hbm.at[pl.ds(base, CHUNK)],
                             semo[b])

        def wait_out(b):
            pltpu.make_async_copy(rows.at[b], out_hbm.at[pl.ds(wbase, CHUNK)],
                                  semo[b]).wait()

        fire_energy(0, 0)

        @pl.loop(0, n_outer)
        def _outer(o):
            for b in range(NBUF):
                c = o * NBUF + b
                pb = 1 - b

                @pl.when(o > 0)
                def _reclaim():
                    wait_out(b)

                wait_energy(b)
                if b == 0:
                    fire_energy(pb, c + 1)
                else:
                    @pl.when(o < n_outer - 1)
                    def _prefetch():
                        fire_energy(pb, c + 1)

                fire_gathers(b, c)

                if b == 0:
                    @pl.when(o > 0)
                    def _drain_prev():
                        wait_gathers(pb)
                        fire_out(pb, c - 1)
                else:
                    wait_gathers(pb)
                    fire_out(pb, c - 1)

        wait_gathers(1)
        fire_out(1, n_chunks - 1)
        wait_out(0)
        wait_out(1)

    return k


def _tc_transpose(nrows, ncol):
    ncol2 = ncol // 2
    rows_per_blk = IBLK * ncol * HIDDEN_DIM // 128

    def body(x_ref, o_ref):
        x = x_ref[...]
        w = x.reshape(IBLK, ncol2 * 128)
        o_ref[...] = w.T.reshape(ncol, HIDDEN_DIM, IBLK)

    return pl.pallas_call(
        body,
        grid=(nrows // IBLK,),
        in_specs=[pl.BlockSpec((rows_per_blk, 128), lambda s: (s, 0))],
        out_specs=pl.BlockSpec((ncol, HIDDEN_DIM, IBLK), lambda s: (0, 0, s)),
        out_shape=jax.ShapeDtypeStruct((ncol, HIDDEN_DIM, nrows), jnp.float32),
    )


def kernel(energy, table, energy_min, energy_max):
    nrows, ncol = energy.shape
    total = nrows * ncol
    b_per_w = total // NW
    n_chunks = b_per_w // CHUNK
    assert b_per_w * NW == total and n_chunks * CHUNK == b_per_w
    assert n_chunks % NBUF == 0

    scale = jnp.float32(N_BINS - 1) / (energy_max - energy_min + jnp.float32(1e-8))
    emin16 = jnp.full((L,), energy_min, dtype=jnp.float32)
    scale16 = jnp.full((L,), scale, dtype=jnp.float32)

    flat = _sc_gather(total, b_per_w, n_chunks)(
        energy.reshape(-1), table, emin16, scale16)
    twod = flat.reshape(total * HIDDEN_DIM // 128, 128)
    b = _tc_transpose(nrows, ncol)(twod)
    return b.transpose(2, 0, 1)
